# baseline (speedup 1.0000x reference)
import functools

import jax
import jax.numpy as jnp
from jax import lax
from jax.experimental import pallas as pl
from jax.experimental.pallas import tpu as pltpu
from jax.experimental.pallas import tpu_sc as plsc

N_NODES = 10000
N_EDGES = 320000
D_FEAT = 128

NC = 2
NS = 16
NW = NC * NS
EPW = N_EDGES // NW
ECHUNK = 80
NCHUNK = EPW // ECHUNK
NPAIR = (NCHUNK - 1) // 2
ZROWS = ECHUNK
NZBLK = N_NODES // ZROWS
WROWS = 200
NWBLK = N_NODES // WROWS


def _sc_body(dst_hbm, src_hbm, a_hbm, x_hbm, out_hbm,
             a_all, sr0, sr1, sr2, sr3, d0, d1, d2, d3, g0, g1, sbuf,
             acc, hsem, isem0, isem1, gsem0, gsem1):
    c = lax.axis_index("c")
    s = lax.axis_index("s")
    wid = c * NS + s
    base = wid * EPW

    srbuf = (sr0, sr1, sr2, sr3)
    dbuf = (d0, d1, d2, d3)
    gbuf = (g0, g1)
    isem = (isem0, isem1)
    gsem = (gsem0, gsem1)

    ha = pltpu.async_copy(a_hbm.at[pl.ds(base, EPW)], a_all, hsem)

    def zrow(i, carry):
        for j in range(D_FEAT // 16):
            g0[i, pl.ds(j * 16, 16)] = jnp.zeros((16,), jnp.float32)
        return carry
    lax.fori_loop(0, ZROWS, zrow, 0)
    for b in range((NZBLK + NS - 1) // NS):
        blk = b * NS + s

        @pl.when(blk < NZBLK)
        def _():
            pltpu.sync_copy(g0, acc.at[pl.ds(blk * ZROWS, ZROWS)])
    ha.wait()
    plsc.subcore_barrier()

    def fetch_src(ci, q4):
        pltpu.async_copy(src_hbm.at[pl.ds(base + ci * ECHUNK, ECHUNK)],
                         srbuf[q4], isem[q4 % 2])

    def wait_src(q4):
        pltpu.make_async_copy(src_hbm.at[pl.ds(0, ECHUNK)],
                              srbuf[q4], isem[q4 % 2]).wait()

    def start_gather(ci, q4):
        p = q4 % 2
        pltpu.async_copy(dst_hbm.at[pl.ds(base + ci * ECHUNK, ECHUNK)],
                         dbuf[q4], gsem[p])
        pltpu.async_copy(x_hbm.at[srbuf[q4]], gbuf[p], gsem[p])

    def wait_gather(q4):
        p = q4 % 2
        pltpu.make_async_copy(dst_hbm.at[pl.ds(0, ECHUNK)],
                              dbuf[q4], gsem[p]).wait()
        pltpu.make_async_copy(x_hbm.at[srbuf[q4]], gbuf[p], gsem[p]).wait()

    def scale(ci, q4):
        g_r = gbuf[q4 % 2]

        def gbody(g, gcarry):
            av16 = a_all[pl.ds(ci * ECHUNK + g * 16, 16)]
            for l in range(16):
                a = av16[l]
                e = g * 16 + l
                for j in range(D_FEAT // 16):
                    sl = pl.ds(j * 16, 16)
                    sbuf[e, sl] = g_r[e, sl] * a
            return gcarry
        lax.fori_loop(0, ECHUNK // 16, gbody, 0)

    def scatter_add(q4):
        pltpu.sync_copy(sbuf, acc.at[dbuf[q4]], add=True)

    def chunk_step(i, q4):
        wait_gather(q4)
        scale(i, q4)
        nxt = (q4 + 2) % 4

        @pl.when(i + 2 < NCHUNK)
        def _():
            wait_src(nxt)
            start_gather(i + 2, nxt)

        @pl.when(i + 4 < NCHUNK)
        def _():
            fetch_src(i + 4, q4)
        scatter_add(q4)

    fetch_src(0, 0)
    fetch_src(1, 1)
    wait_src(0)
    start_gather(0, 0)
    wait_src(1)
    start_gather(1, 1)
    fetch_src(2, 2)
    fetch_src(3, 3)

    def pair_body(k, carry):
        chunk_step(2 * k, (2 * k) % 4)
        chunk_step(2 * k + 1, (2 * k + 1) % 4)
        return carry

    def quad_body(m, carry):
        chunk_step(4 * m, 0)
        chunk_step(4 * m + 1, 1)
        chunk_step(4 * m + 2, 2)
        chunk_step(4 * m + 3, 3)
        return carry
    lax.fori_loop(0, NCHUNK // 4, quad_body, 0)
    chunk_step(NCHUNK - 1, 0)

    plsc.subcore_barrier()
    for b in range((NWBLK + NS - 1) // NS):
        blk = b * NS + s

        @pl.when(blk < NWBLK)
        def _():
            r = blk * WROWS
            pltpu.sync_copy(acc.at[pl.ds(r, WROWS)],
                            out_hbm.at[c, pl.ds(r, WROWS)])


def _combine_body(p_ref, o_ref):
    o_ref[...] = p_ref[0] + p_ref[1]


def kernel(edge_index, A_values, X):
    mesh = plsc.VectorSubcoreMesh(core_axis_name="c", subcore_axis_name="s")
    sc_call = functools.partial(
        pl.kernel,
        mesh=mesh,
        out_type=jax.ShapeDtypeStruct((NC, N_NODES, D_FEAT), jnp.float32),
        scratch_types=[
            pltpu.VMEM((EPW,), jnp.float32),
            pltpu.VMEM((ECHUNK,), jnp.int32),
            pltpu.VMEM((ECHUNK,), jnp.int32),
            pltpu.VMEM((ECHUNK,), jnp.int32),
            pltpu.VMEM((ECHUNK,), jnp.int32),
            pltpu.VMEM((ECHUNK,), jnp.int32),
            pltpu.VMEM((ECHUNK,), jnp.int32),
            pltpu.VMEM((ECHUNK,), jnp.int32),
            pltpu.VMEM((ECHUNK,), jnp.int32),
            pltpu.VMEM((ECHUNK, D_FEAT), jnp.float32),
            pltpu.VMEM((ECHUNK, D_FEAT), jnp.float32),
            pltpu.VMEM((ECHUNK, D_FEAT), jnp.float32),
            pltpu.VMEM_SHARED((N_NODES, D_FEAT), jnp.float32),
            pltpu.SemaphoreType.DMA,
            pltpu.SemaphoreType.DMA,
            pltpu.SemaphoreType.DMA,
            pltpu.SemaphoreType.DMA,
            pltpu.SemaphoreType.DMA,
        ],
    )(_sc_body)
    partials = sc_call(edge_index[0], edge_index[1], A_values, X)

    combine = pl.pallas_call(
        _combine_body,
        out_shape=jax.ShapeDtypeStruct((N_NODES, D_FEAT), jnp.float32),
        grid=(10,),
        in_specs=[pl.BlockSpec((NC, N_NODES // 10, D_FEAT), lambda i: (0, i, 0))],
        out_specs=pl.BlockSpec((N_NODES // 10, D_FEAT), lambda i: (i, 0)),
    )
    return combine(partials)
